# baseline (speedup 1.0000x reference)
import functools

import jax
import jax.numpy as jnp
from jax import lax
from jax.experimental import pallas as pl
from jax.experimental.pallas import tpu as pltpu
from jax.experimental.pallas import tpu_sc as plsc

B, L, M, D, F, K, NB = 16, 512, 2048, 256, 256, 3, 256
LP = L + 1
HALF = M // 2
NCHUNK = HALF // 16
ROWS = 128
NGRP = HALF // ROWS
W = D // 2
NBUF = 4


def _sc_body(xpad, dur, ptgt, etgt, pbkt, ebkt, pemb, eemb,
             xm_out, pemb_out, eemb_out,
             dur_v, csum_v, idx_v, pidx_v, eidx_v, tgt_v, bkt_v,
             bufs, gsems, wsems):
  cid = lax.axis_index("c")
  sid = lax.axis_index("s")
  wid = sid * 2 + cid
  b = wid // 2
  half = wid % 2
  mbase = half * HALF
  rowbase = b * M + mbase

  pltpu.sync_copy(dur.at[pl.ds(b * L, L)], dur_v.at[pl.ds(0, L)])
  lanes = lax.iota(jnp.int32, 16)
  carry = jnp.int32(0)
  for i in range(L // 16):
    d = dur_v[pl.ds(i * 16, 16)]
    csum_v[pl.ds(i * 16, 16)] = plsc.cumsum(d) + carry
    carry = carry + jnp.sum(d)
  big = jnp.full((16,), jnp.int32(1 << 30))
  for i in range(L // 16, 2 * L // 16):
    csum_v[pl.ds(i * 16, 16)] = big

  def seg_chunk(i, _):
    m_vec = mbase + i * 16 + lanes
    pos = jnp.zeros((16,), jnp.int32)
    for k in (512, 256, 128, 64, 32, 16, 8, 4, 2, 1):
      cand = pos + k
      vals = plsc.load_gather(csum_v, (cand - 1,))
      pos = jnp.where(vals <= m_vec, cand, pos)
    idx_v[pl.ds(i * 16, 16)] = b * LP + pos
    return 0

  lax.fori_loop(0, NCHUNK, seg_chunk, 0, unroll=4)

  def bucketize(tgt_hbm, bkt_hbm, out_idx):
    pltpu.sync_copy(bkt_hbm, bkt_v)
    pltpu.sync_copy(tgt_hbm.at[pl.ds(b * M + mbase, HALF)], tgt_v)

    def bkt_chunk(i, _):
      t = tgt_v[pl.ds(i * 16, 16)]
      pos = jnp.zeros((16,), jnp.int32)
      for k in (128, 64, 32, 16, 8, 4, 2, 1):
        cand = pos + k
        vals = plsc.load_gather(bkt_v, (cand - 1,))
        pos = jnp.where(vals < t, cand, pos)
      out_idx[pl.ds(i * 16, 16)] = pos
      return 0

    lax.fori_loop(0, NCHUNK, bkt_chunk, 0, unroll=4)

  bucketize(ptgt, pbkt, pidx_v)
  bucketize(etgt, ebkt, eidx_v)

  tasks = []
  for g in range(NGRP):
    tasks.append((xpad, idx_v, xm_out, g))
  for g in range(NGRP):
    tasks.append((pemb, pidx_v, pemb_out, g))
  for g in range(NGRP):
    tasks.append((eemb, eidx_v, eemb_out, g))

  T = len(tasks)
  gh = [None] * NBUF
  wh = [None] * NBUF
  for t in range(T + 1):
    if t < T:
      s = t % NBUF
      if wh[s] is not None:
        wh[s].wait()
      table, idxref, out, g = tasks[t]
      gh[s] = pltpu.async_copy(
          table.at[idxref.at[pl.ds(g * ROWS, ROWS)]], bufs.at[s], gsems[s])
    if t >= 1:
      p = (t - 1) % NBUF
      gh[p].wait()
      _, _, pout, pg = tasks[t - 1]
      wh[p] = pltpu.async_copy(
          bufs.at[p], pout.at[pl.ds(rowbase + pg * ROWS, ROWS)], wsems[p])
  for s in range(NBUF):
    if wh[s] is not None:
      wh[s].wait()


def _sc_lr_embed(xpad, dur_flat, ptgt_flat, etgt_flat, pbkt_pad, ebkt_pad,
                 pemb, eemb):
  mesh = plsc.VectorSubcoreMesh(core_axis_name="c", subcore_axis_name="s")
  f32 = jnp.float32
  run = pl.kernel(
      _sc_body,
      out_type=[jax.ShapeDtypeStruct((B * M, W), f32) for _ in range(3)],
      mesh=mesh,
      compiler_params=pltpu.CompilerParams(needs_layout_passes=False),
      scratch_types=[
          pltpu.VMEM((L,), jnp.int32),
          pltpu.VMEM((2 * L,), jnp.int32),
          pltpu.VMEM((HALF,), jnp.int32),
          pltpu.VMEM((HALF,), jnp.int32),
          pltpu.VMEM((HALF,), jnp.int32),
          pltpu.VMEM((HALF,), f32),
          pltpu.VMEM((NB,), f32),
          pltpu.VMEM((NBUF, ROWS, W), f32),
          [pltpu.SemaphoreType.DMA] * NBUF,
          [pltpu.SemaphoreType.DMA] * NBUF,
      ],
  )
  return run(xpad, dur_flat, ptgt_flat, etgt_flat, pbkt_pad, ebkt_pad,
             pemb, eemb)


def _conv_ln_relu(x, wk, bias, g, bb):
  z = jnp.zeros((1, x.shape[1]), x.dtype)
  xdn = jnp.concatenate([z, x[:-1]], axis=0)
  xup = jnp.concatenate([x[1:], z], axis=0)
  y = (jnp.dot(xdn, wk[0], preferred_element_type=jnp.float32)
       + jnp.dot(x, wk[1], preferred_element_type=jnp.float32)
       + jnp.dot(xup, wk[2], preferred_element_type=jnp.float32)
       + bias[0][None, :])
  m = jnp.mean(y, axis=-1, keepdims=True)
  v = jnp.mean((y - m) ** 2, axis=-1, keepdims=True)
  h = (y - m) * lax.rsqrt(v + 1e-5) * g[0][None, :] + bb[0][None, :]
  return jnp.maximum(h, 0.0)


def _pred_tail(h, lw, lb):
  return jnp.maximum(jnp.sum(h * lw[0][None, :], axis=-1) + lb[0, 0], 0.0)


def _dur_body(x_ref, wk1, b1, g1, bb1, wk2, b2, g2, bb2, lw, lb, out_ref):
  h = _conv_ln_relu(x_ref[0], wk1, b1, g1, bb1)
  h = _conv_ln_relu(h, wk2, b2, g2, bb2)
  out_ref[0, 0, :] = _pred_tail(h, lw, lb)


def _ce_body(mlen_ref, xm_ref, pe_ref, ee_ref,
             pwk1, pb1, pg1, pbb1, pwk2, pb2, pg2, pbb2, plw, plb,
             ewk1, eb1, eg1, ebb1, ewk2, eb2, eg2, ebb2, elw, elb,
             pp_ref, ep_ref, fin_ref):
  frames = lax.broadcasted_iota(jnp.int32, (M, 1), 0)
  xm = jnp.where(frames < mlen_ref[0], xm_ref[0].astype(jnp.float32), 0.0)
  h = _conv_ln_relu(xm, pwk1, pb1, pg1, pbb1)
  h = _conv_ln_relu(h, pwk2, pb2, pg2, pbb2)
  pp_ref[0, 0, :] = _pred_tail(h, plw, plb)
  x2 = xm + pe_ref[0].astype(jnp.float32)
  h = _conv_ln_relu(x2, ewk1, eb1, eg1, ebb1)
  h = _conv_ln_relu(h, ewk2, eb2, eg2, ebb2)
  ep_ref[0, 0, :] = _pred_tail(h, elw, elb)
  fin_ref[0] = x2 + ee_ref[0].astype(jnp.float32)


def _prep(p):
  return (jnp.transpose(p['w1'], (2, 1, 0)), p['b1'][None, :],
          p['g1'][None, :], p['bb1'][None, :],
          jnp.transpose(p['w2'], (2, 1, 0)), p['b2'][None, :],
          p['g2'][None, :], p['bb2'][None, :],
          p['lw'], p['lb'][None, :])


def _wspecs():
  full = lambda shape: pl.BlockSpec(shape, lambda b: (0,) * len(shape))
  return [full((K, D, F)), full((1, F)), full((1, F)), full((1, F)),
          full((K, F, F)), full((1, F)), full((1, F)), full((1, F)),
          full((1, F)), full((1, 1))]


def _dur_pred(x, p):
  seq = pl.BlockSpec((1, L, D), lambda b: (b, 0, 0))
  out = pl.pallas_call(
      _dur_body,
      grid=(B,),
      in_specs=[seq] + _wspecs(),
      out_specs=pl.BlockSpec((1, 1, L), lambda b: (b, 0, 0)),
      out_shape=jax.ShapeDtypeStruct((B, 1, L), jnp.float32),
  )(x, *_prep(p))
  return out.reshape(B, L)


def _pitch_energy(max_len, xm, pemb, eemb, pp, ep):
  seq = pl.BlockSpec((1, M, D), lambda b: (b, 0, 0))
  pred = pl.BlockSpec((1, 1, M), lambda b: (b, 0, 0))
  sspec = pl.BlockSpec(memory_space=pltpu.SMEM)
  ppd, epd, fin = pl.pallas_call(
      _ce_body,
      grid=(B,),
      in_specs=[sspec, seq, seq, seq] + _wspecs() + _wspecs(),
      out_specs=[pred, pred, seq],
      out_shape=[jax.ShapeDtypeStruct((B, 1, M), jnp.float32),
                 jax.ShapeDtypeStruct((B, 1, M), jnp.float32),
                 jax.ShapeDtypeStruct((B, M, D), jnp.float32)],
  )(jnp.asarray(max_len, jnp.int32).reshape(1), xm, pemb, eemb,
    *_prep(pp), *_prep(ep))
  return ppd.reshape(B, M), epd.reshape(B, M), fin


def _pack_bf16(a):
  b = a.astype(jnp.bfloat16).reshape(a.shape[0], W, 2)
  return lax.bitcast_convert_type(b, jnp.float32)


def _unpack_bf16(a):
  return lax.bitcast_convert_type(a, jnp.bfloat16).reshape(B, M, D)


def kernel(x, dur_target, pitch_target, energy_target, max_len, mask, params,
           pitch_bucket, energy_bucket):
  f32 = jnp.float32
  xpad = jnp.concatenate([x, jnp.zeros((B, 1, D), f32)], axis=1)
  xpad = _pack_bf16(xpad.reshape(B * LP, D))
  inf = jnp.full((1,), jnp.inf, f32)
  pbkt_pad = jnp.concatenate([pitch_bucket, inf])
  ebkt_pad = jnp.concatenate([energy_bucket, inf])

  xm, pemb_rows, eemb_rows = _sc_lr_embed(
      xpad, dur_target.reshape(-1), pitch_target.reshape(-1),
      energy_target.reshape(-1), pbkt_pad, ebkt_pad,
      _pack_bf16(params['pitch_emb']), _pack_bf16(params['energy_emb']))

  dur_pred = _dur_pred(x, params['dur'])
  pitch_pred, energy_pred, final = _pitch_energy(
      max_len, _unpack_bf16(xm), _unpack_bf16(pemb_rows),
      _unpack_bf16(eemb_rows), params['pitch'], params['energy'])
  return (final, dur_pred, pitch_pred, energy_pred)


# speedup vs baseline: 1.7367x; 1.7367x over previous
import functools

import jax
import jax.numpy as jnp
from jax import lax
from jax.experimental import pallas as pl
from jax.experimental.pallas import tpu as pltpu
from jax.experimental.pallas import tpu_sc as plsc

B, L, M, D, F, K, NB = 16, 512, 2048, 256, 256, 3, 256
LP = L + 1
HALF = M // 2
NCHUNK = HALF // 16
ROWS = 64
NGRP = HALF // ROWS
W = D
NBUF = 4


def _sc_body(xpad, dur, ptgt, etgt, pbkt, ebkt, pemb, eemb,
             xm_out, pemb_out, eemb_out,
             dur_v, csum_v, idx_v, pidx_v, eidx_v, tgt_v, bkt_v,
             bufs, gsems, wsems):
  cid = lax.axis_index("c")
  sid = lax.axis_index("s")
  wid = sid * 2 + cid
  b = wid // 2
  half = wid % 2
  mbase = half * HALF
  rowbase = b * M + mbase

  pltpu.sync_copy(dur.at[pl.ds(b * L, L)], dur_v.at[pl.ds(0, L)])
  lanes = lax.iota(jnp.int32, 16)
  carry = jnp.int32(0)
  for i in range(L // 16):
    d = dur_v[pl.ds(i * 16, 16)]
    csum_v[pl.ds(i * 16, 16)] = plsc.cumsum(d) + carry
    carry = carry + jnp.sum(d)
  big = jnp.full((16,), jnp.int32(1 << 30))
  for i in range(L // 16, 2 * L // 16):
    csum_v[pl.ds(i * 16, 16)] = big

  def seg_chunk(i, _):
    m_vec = mbase + i * 16 + lanes
    pos = jnp.zeros((16,), jnp.int32)
    for k in (512, 256, 128, 64, 32, 16, 8, 4, 2, 1):
      cand = pos + k
      vals = plsc.load_gather(csum_v, (cand - 1,))
      pos = jnp.where(vals <= m_vec, cand, pos)
    idx_v[pl.ds(i * 16, 16)] = b * LP + pos
    return 0

  lax.fori_loop(0, NCHUNK, seg_chunk, 0, unroll=4)

  def bucketize(tgt_hbm, bkt_hbm, out_idx):
    pltpu.sync_copy(bkt_hbm, bkt_v)
    pltpu.sync_copy(tgt_hbm.at[pl.ds(b * M + mbase, HALF)], tgt_v)

    def bkt_chunk(i, _):
      t = tgt_v[pl.ds(i * 16, 16)]
      pos = jnp.zeros((16,), jnp.int32)
      for k in (128, 64, 32, 16, 8, 4, 2, 1):
        cand = pos + k
        vals = plsc.load_gather(bkt_v, (cand - 1,))
        pos = jnp.where(vals < t, cand, pos)
      out_idx[pl.ds(i * 16, 16)] = pos
      return 0

    lax.fori_loop(0, NCHUNK, bkt_chunk, 0, unroll=4)

  bucketize(ptgt, pbkt, pidx_v)
  bucketize(etgt, ebkt, eidx_v)

  tasks = []
  for g in range(NGRP):
    tasks.append((xpad, idx_v, xm_out, g))
  for g in range(NGRP):
    tasks.append((pemb, pidx_v, pemb_out, g))
  for g in range(NGRP):
    tasks.append((eemb, eidx_v, eemb_out, g))

  T = len(tasks)
  gh = [None] * NBUF
  wh = [None] * NBUF
  for t in range(T + 1):
    if t < T:
      s = t % NBUF
      if wh[s] is not None:
        wh[s].wait()
      table, idxref, out, g = tasks[t]
      gh[s] = pltpu.async_copy(
          table.at[idxref.at[pl.ds(g * ROWS, ROWS)]], bufs.at[s], gsems[s])
    if t >= 1:
      p = (t - 1) % NBUF
      gh[p].wait()
      _, _, pout, pg = tasks[t - 1]
      wh[p] = pltpu.async_copy(
          bufs.at[p], pout.at[pl.ds(rowbase + pg * ROWS, ROWS)], wsems[p])
  for s in range(NBUF):
    if wh[s] is not None:
      wh[s].wait()


def _sc_lr_embed(xpad, dur_flat, ptgt_flat, etgt_flat, pbkt_pad, ebkt_pad,
                 pemb, eemb):
  mesh = plsc.VectorSubcoreMesh(core_axis_name="c", subcore_axis_name="s")
  f32 = jnp.float32
  run = pl.kernel(
      _sc_body,
      out_type=[jax.ShapeDtypeStruct((B * M, W), f32) for _ in range(3)],
      mesh=mesh,
      compiler_params=pltpu.CompilerParams(needs_layout_passes=False),
      scratch_types=[
          pltpu.VMEM((L,), jnp.int32),
          pltpu.VMEM((2 * L,), jnp.int32),
          pltpu.VMEM((HALF,), jnp.int32),
          pltpu.VMEM((HALF,), jnp.int32),
          pltpu.VMEM((HALF,), jnp.int32),
          pltpu.VMEM((HALF,), f32),
          pltpu.VMEM((NB,), f32),
          pltpu.VMEM((NBUF, ROWS, W), f32),
          [pltpu.SemaphoreType.DMA] * NBUF,
          [pltpu.SemaphoreType.DMA] * NBUF,
      ],
  )
  return run(xpad, dur_flat, ptgt_flat, etgt_flat, pbkt_pad, ebkt_pad,
             pemb, eemb)


def _conv_ln_relu(x, wk, bias, g, bb):
  z = jnp.zeros((1, x.shape[1]), x.dtype)
  xdn = jnp.concatenate([z, x[:-1]], axis=0)
  xup = jnp.concatenate([x[1:], z], axis=0)
  y = (jnp.dot(xdn, wk[0], preferred_element_type=jnp.float32)
       + jnp.dot(x, wk[1], preferred_element_type=jnp.float32)
       + jnp.dot(xup, wk[2], preferred_element_type=jnp.float32)
       + bias[0][None, :])
  m = jnp.mean(y, axis=-1, keepdims=True)
  v = jnp.mean((y - m) ** 2, axis=-1, keepdims=True)
  h = (y - m) * lax.rsqrt(v + 1e-5) * g[0][None, :] + bb[0][None, :]
  return jnp.maximum(h, 0.0)


def _pred_tail(h, lw, lb):
  return jnp.maximum(jnp.sum(h * lw[0][None, :], axis=-1) + lb[0, 0], 0.0)


def _dur_body(x_ref, wk1, b1, g1, bb1, wk2, b2, g2, bb2, lw, lb, out_ref):
  h = _conv_ln_relu(x_ref[0], wk1, b1, g1, bb1)
  h = _conv_ln_relu(h, wk2, b2, g2, bb2)
  out_ref[0, 0, :] = _pred_tail(h, lw, lb)


def _ce_body(mlen_ref, xm_ref, pe_ref, ee_ref,
             pwk1, pb1, pg1, pbb1, pwk2, pb2, pg2, pbb2, plw, plb,
             ewk1, eb1, eg1, ebb1, ewk2, eb2, eg2, ebb2, elw, elb,
             pp_ref, ep_ref, fin_ref):
  frames = lax.broadcasted_iota(jnp.int32, (M, 1), 0)
  xm = jnp.where(frames < mlen_ref[0], xm_ref[0], 0.0)
  h = _conv_ln_relu(xm, pwk1, pb1, pg1, pbb1)
  h = _conv_ln_relu(h, pwk2, pb2, pg2, pbb2)
  pp_ref[0, 0, :] = _pred_tail(h, plw, plb)
  x2 = xm + pe_ref[0]
  h = _conv_ln_relu(x2, ewk1, eb1, eg1, ebb1)
  h = _conv_ln_relu(h, ewk2, eb2, eg2, ebb2)
  ep_ref[0, 0, :] = _pred_tail(h, elw, elb)
  fin_ref[0] = x2 + ee_ref[0]


def _prep(p):
  return (jnp.transpose(p['w1'], (2, 1, 0)), p['b1'][None, :],
          p['g1'][None, :], p['bb1'][None, :],
          jnp.transpose(p['w2'], (2, 1, 0)), p['b2'][None, :],
          p['g2'][None, :], p['bb2'][None, :],
          p['lw'], p['lb'][None, :])


def _wspecs():
  full = lambda shape: pl.BlockSpec(shape, lambda b: (0,) * len(shape))
  return [full((K, D, F)), full((1, F)), full((1, F)), full((1, F)),
          full((K, F, F)), full((1, F)), full((1, F)), full((1, F)),
          full((1, F)), full((1, 1))]


def _dur_pred(x, p):
  seq = pl.BlockSpec((1, L, D), lambda b: (b, 0, 0))
  out = pl.pallas_call(
      _dur_body,
      grid=(B,),
      in_specs=[seq] + _wspecs(),
      out_specs=pl.BlockSpec((1, 1, L), lambda b: (b, 0, 0)),
      out_shape=jax.ShapeDtypeStruct((B, 1, L), jnp.float32),
  )(x, *_prep(p))
  return out.reshape(B, L)


def _pitch_energy(max_len, xm, pemb, eemb, pp, ep):
  seq = pl.BlockSpec((1, M, D), lambda b: (b, 0, 0))
  pred = pl.BlockSpec((1, 1, M), lambda b: (b, 0, 0))
  sspec = pl.BlockSpec(memory_space=pltpu.SMEM)
  ppd, epd, fin = pl.pallas_call(
      _ce_body,
      grid=(B,),
      in_specs=[sspec, seq, seq, seq] + _wspecs() + _wspecs(),
      out_specs=[pred, pred, seq],
      out_shape=[jax.ShapeDtypeStruct((B, 1, M), jnp.float32),
                 jax.ShapeDtypeStruct((B, 1, M), jnp.float32),
                 jax.ShapeDtypeStruct((B, M, D), jnp.float32)],
  )(jnp.asarray(max_len, jnp.int32).reshape(1), xm, pemb, eemb,
    *_prep(pp), *_prep(ep))
  return ppd.reshape(B, M), epd.reshape(B, M), fin


def _pack_bf16(a):
  b = a.astype(jnp.bfloat16).reshape(a.shape[0], W, 2)
  return lax.bitcast_convert_type(b, jnp.float32)


def _unpack_bf16(a):
  return lax.bitcast_convert_type(a, jnp.bfloat16).reshape(B, M, D)


def kernel(x, dur_target, pitch_target, energy_target, max_len, mask, params,
           pitch_bucket, energy_bucket):
  f32 = jnp.float32
  xpad = jnp.concatenate([x, jnp.zeros((B, 1, D), f32)], axis=1)
  xpad = xpad.reshape(B * LP, D)
  inf = jnp.full((1,), jnp.inf, f32)
  pbkt_pad = jnp.concatenate([pitch_bucket, inf])
  ebkt_pad = jnp.concatenate([energy_bucket, inf])

  xm, pemb_rows, eemb_rows = _sc_lr_embed(
      xpad, dur_target.reshape(-1), pitch_target.reshape(-1),
      energy_target.reshape(-1), pbkt_pad, ebkt_pad,
      params['pitch_emb'], params['energy_emb'])

  dur_pred = _dur_pred(x, params['dur'])
  pitch_pred, energy_pred, final = _pitch_energy(
      max_len, xm.reshape(B, M, D), pemb_rows.reshape(B, M, D),
      eemb_rows.reshape(B, M, D), params['pitch'], params['energy'])
  return (final, dur_pred, pitch_pred, energy_pred)


# speedup vs baseline: 1.7526x; 1.0091x over previous
import functools

import jax
import jax.numpy as jnp
from jax import lax
from jax.experimental import pallas as pl
from jax.experimental.pallas import tpu as pltpu
from jax.experimental.pallas import tpu_sc as plsc

B, L, M, D, F, K, NB = 16, 512, 2048, 256, 256, 3, 256
LP = L + 1
HALF = M // 2
NCHUNK = HALF // 16
ROWS = 64
NGRP = HALF // ROWS
W = D
NBUF = 4


def _sc_body(xpad, dur, ptgt, etgt, pbkt, ebkt,
             xm_out, pidx_out, eidx_out,
             dur_v, csum_v, idx_v, pidx_v, eidx_v, tgt_v, bkt_v,
             bufs, gsems, wsems):
  cid = lax.axis_index("c")
  sid = lax.axis_index("s")
  wid = sid * 2 + cid
  b = wid // 2
  half = wid % 2
  mbase = half * HALF
  rowbase = b * M + mbase

  pltpu.sync_copy(dur.at[pl.ds(b * L, L)], dur_v.at[pl.ds(0, L)])
  lanes = lax.iota(jnp.int32, 16)
  carry = jnp.int32(0)
  for i in range(L // 16):
    d = dur_v[pl.ds(i * 16, 16)]
    csum_v[pl.ds(i * 16, 16)] = plsc.cumsum(d) + carry
    carry = carry + jnp.sum(d)
  big = jnp.full((16,), jnp.int32(1 << 30))
  for i in range(L // 16, 2 * L // 16):
    csum_v[pl.ds(i * 16, 16)] = big

  def seg_chunk(i, _):
    m_vec = mbase + i * 16 + lanes
    pos = jnp.zeros((16,), jnp.int32)
    for k in (512, 256, 128, 64, 32, 16, 8, 4, 2, 1):
      cand = pos + k
      vals = plsc.load_gather(csum_v, (cand - 1,))
      pos = jnp.where(vals <= m_vec, cand, pos)
    idx_v[pl.ds(i * 16, 16)] = b * LP + pos
    return 0

  lax.fori_loop(0, NCHUNK, seg_chunk, 0, unroll=4)

  def bucketize(tgt_hbm, bkt_hbm, out_idx):
    pltpu.sync_copy(bkt_hbm, bkt_v)
    pltpu.sync_copy(tgt_hbm.at[pl.ds(b * M + mbase, HALF)], tgt_v)

    def bkt_chunk(i, _):
      t = tgt_v[pl.ds(i * 16, 16)]
      pos = jnp.zeros((16,), jnp.int32)
      for k in (128, 64, 32, 16, 8, 4, 2, 1):
        cand = pos + k
        vals = plsc.load_gather(bkt_v, (cand - 1,))
        pos = jnp.where(vals < t, cand, pos)
      out_idx[pl.ds(i * 16, 16)] = pos
      return 0

    lax.fori_loop(0, NCHUNK, bkt_chunk, 0, unroll=4)

  bucketize(ptgt, pbkt, pidx_v)
  bucketize(etgt, ebkt, eidx_v)
  pltpu.sync_copy(pidx_v, pidx_out.at[pl.ds(rowbase, HALF)])
  pltpu.sync_copy(eidx_v, eidx_out.at[pl.ds(rowbase, HALF)])

  tasks = [(xpad, idx_v, xm_out, g) for g in range(NGRP)]

  T = len(tasks)
  gh = [None] * NBUF
  wh = [None] * NBUF
  for t in range(T + 1):
    if t < T:
      s = t % NBUF
      if wh[s] is not None:
        wh[s].wait()
      table, idxref, out, g = tasks[t]
      gh[s] = pltpu.async_copy(
          table.at[idxref.at[pl.ds(g * ROWS, ROWS)]], bufs.at[s], gsems[s])
    if t >= 1:
      p = (t - 1) % NBUF
      gh[p].wait()
      _, _, pout, pg = tasks[t - 1]
      wh[p] = pltpu.async_copy(
          bufs.at[p], pout.at[pl.ds(rowbase + pg * ROWS, ROWS)], wsems[p])
  for s in range(NBUF):
    if wh[s] is not None:
      wh[s].wait()


def _sc_lr_embed(xpad, dur_flat, ptgt_flat, etgt_flat, pbkt_pad, ebkt_pad):
  mesh = plsc.VectorSubcoreMesh(core_axis_name="c", subcore_axis_name="s")
  f32 = jnp.float32
  run = pl.kernel(
      _sc_body,
      out_type=[jax.ShapeDtypeStruct((B * M, W), f32),
                jax.ShapeDtypeStruct((B * M,), jnp.int32),
                jax.ShapeDtypeStruct((B * M,), jnp.int32)],
      mesh=mesh,
      compiler_params=pltpu.CompilerParams(needs_layout_passes=False),
      scratch_types=[
          pltpu.VMEM((L,), jnp.int32),
          pltpu.VMEM((2 * L,), jnp.int32),
          pltpu.VMEM((HALF,), jnp.int32),
          pltpu.VMEM((HALF,), jnp.int32),
          pltpu.VMEM((HALF,), jnp.int32),
          pltpu.VMEM((HALF,), f32),
          pltpu.VMEM((NB,), f32),
          pltpu.VMEM((NBUF, ROWS, W), f32),
          [pltpu.SemaphoreType.DMA] * NBUF,
          [pltpu.SemaphoreType.DMA] * NBUF,
      ],
  )
  return run(xpad, dur_flat, ptgt_flat, etgt_flat, pbkt_pad, ebkt_pad)


def _conv_ln_relu(x, wk, bias, g, bb):
  x = x.astype(jnp.bfloat16)
  z = jnp.zeros((1, x.shape[1]), x.dtype)
  xdn = jnp.concatenate([z, x[:-1]], axis=0)
  xup = jnp.concatenate([x[1:], z], axis=0)
  y = (jnp.dot(xdn, wk[0], preferred_element_type=jnp.float32)
       + jnp.dot(x, wk[1], preferred_element_type=jnp.float32)
       + jnp.dot(xup, wk[2], preferred_element_type=jnp.float32)
       + bias[0][None, :])
  m = jnp.mean(y, axis=-1, keepdims=True)
  v = jnp.mean((y - m) ** 2, axis=-1, keepdims=True)
  h = (y - m) * lax.rsqrt(v + 1e-5) * g[0][None, :] + bb[0][None, :]
  return jnp.maximum(h, 0.0)


def _pred_tail(h, lw, lb):
  return jnp.maximum(jnp.sum(h * lw[0][None, :], axis=-1) + lb[0, 0], 0.0)


def _dur_body(x_ref, wk1, b1, g1, bb1, wk2, b2, g2, bb2, lw, lb, out_ref):
  h = _conv_ln_relu(x_ref[0], wk1, b1, g1, bb1)
  h = _conv_ln_relu(h, wk2, b2, g2, bb2)
  out_ref[0, 0, :] = _pred_tail(h, lw, lb)


def _emb_rows(idx_col, tab):
  oh = (idx_col == lax.broadcasted_iota(jnp.int32, (M, NB), 1))
  return jnp.dot(oh.astype(jnp.bfloat16), tab.astype(jnp.bfloat16),
                 preferred_element_type=jnp.float32)


def _ce_body(mlen_ref, xm_ref, pidx_ref, eidx_ref, ptab_ref, etab_ref,
             pwk1, pb1, pg1, pbb1, pwk2, pb2, pg2, pbb2, plw, plb,
             ewk1, eb1, eg1, ebb1, ewk2, eb2, eg2, ebb2, elw, elb,
             pp_ref, ep_ref, fin_ref):
  frames = lax.broadcasted_iota(jnp.int32, (M, 1), 0)
  xm = jnp.where(frames < mlen_ref[0], xm_ref[0], 0.0)
  h = _conv_ln_relu(xm, pwk1, pb1, pg1, pbb1)
  h = _conv_ln_relu(h, pwk2, pb2, pg2, pbb2)
  pp_ref[0, 0, :] = _pred_tail(h, plw, plb)
  x2 = xm + _emb_rows(pidx_ref[0], ptab_ref[...])
  h = _conv_ln_relu(x2, ewk1, eb1, eg1, ebb1)
  h = _conv_ln_relu(h, ewk2, eb2, eg2, ebb2)
  ep_ref[0, 0, :] = _pred_tail(h, elw, elb)
  fin_ref[0] = x2 + _emb_rows(eidx_ref[0], etab_ref[...])


def _prep(p):
  bf = jnp.bfloat16
  return (jnp.transpose(p['w1'], (2, 1, 0)).astype(bf), p['b1'][None, :],
          p['g1'][None, :], p['bb1'][None, :],
          jnp.transpose(p['w2'], (2, 1, 0)).astype(bf), p['b2'][None, :],
          p['g2'][None, :], p['bb2'][None, :],
          p['lw'], p['lb'][None, :])


def _wspecs():
  full = lambda shape: pl.BlockSpec(shape, lambda b: (0,) * len(shape))
  return [full((K, D, F)), full((1, F)), full((1, F)), full((1, F)),
          full((K, F, F)), full((1, F)), full((1, F)), full((1, F)),
          full((1, F)), full((1, 1))]


def _dur_pred(x, p):
  seq = pl.BlockSpec((1, L, D), lambda b: (b, 0, 0))
  out = pl.pallas_call(
      _dur_body,
      grid=(B,),
      in_specs=[seq] + _wspecs(),
      out_specs=pl.BlockSpec((1, 1, L), lambda b: (b, 0, 0)),
      out_shape=jax.ShapeDtypeStruct((B, 1, L), jnp.float32),
  )(x, *_prep(p))
  return out.reshape(B, L)


def _pitch_energy(max_len, xm, pidx, eidx, ptab, etab, pp, ep):
  seq = pl.BlockSpec((1, M, D), lambda b: (b, 0, 0))
  idxs = pl.BlockSpec((1, M, 1), lambda b: (b, 0, 0))
  tab = pl.BlockSpec((NB, D), lambda b: (0, 0))
  pred = pl.BlockSpec((1, 1, M), lambda b: (b, 0, 0))
  sspec = pl.BlockSpec(memory_space=pltpu.SMEM)
  ppd, epd, fin = pl.pallas_call(
      _ce_body,
      grid=(B,),
      in_specs=[sspec, seq, idxs, idxs, tab, tab] + _wspecs() + _wspecs(),
      out_specs=[pred, pred, seq],
      out_shape=[jax.ShapeDtypeStruct((B, 1, M), jnp.float32),
                 jax.ShapeDtypeStruct((B, 1, M), jnp.float32),
                 jax.ShapeDtypeStruct((B, M, D), jnp.float32)],
  )(jnp.asarray(max_len, jnp.int32).reshape(1), xm, pidx, eidx, ptab, etab,
    *_prep(pp), *_prep(ep))
  return ppd.reshape(B, M), epd.reshape(B, M), fin


def kernel(x, dur_target, pitch_target, energy_target, max_len, mask, params,
           pitch_bucket, energy_bucket):
  f32 = jnp.float32
  xpad = jnp.concatenate([x, jnp.zeros((B, 1, D), f32)], axis=1)
  xpad = xpad.reshape(B * LP, D)
  inf = jnp.full((1,), jnp.inf, f32)
  pbkt_pad = jnp.concatenate([pitch_bucket, inf])
  ebkt_pad = jnp.concatenate([energy_bucket, inf])

  xm, pidx, eidx = _sc_lr_embed(
      xpad, dur_target.reshape(-1), pitch_target.reshape(-1),
      energy_target.reshape(-1), pbkt_pad, ebkt_pad)

  dur_pred = _dur_pred(x, params['dur'])
  pitch_pred, energy_pred, final = _pitch_energy(
      max_len, xm.reshape(B, M, D), pidx.reshape(B, M, 1),
      eidx.reshape(B, M, 1), params['pitch_emb'], params['energy_emb'],
      params['pitch'], params['energy'])
  return (final, dur_pred, pitch_pred, energy_pred)


# speedup vs baseline: 2.6659x; 1.5211x over previous
import functools

import jax
import jax.numpy as jnp
from jax import lax
from jax.experimental import pallas as pl
from jax.experimental.pallas import tpu as pltpu
from jax.experimental.pallas import tpu_sc as plsc

B, L, M, D, F, K, NB = 16, 512, 2048, 256, 256, 3, 256
LP = L + 1
HALF = M // 2
NCHUNK = HALF // 16
ROWS = 64
NGRP = HALF // ROWS
W = D
NBUF = 4


def _sc_body(xpad, dur, ptgt, etgt, pbkt, ebkt,
             xm_out, pidx_out, eidx_out,
             dur_v, csum_v, idx_v, pidx_v, eidx_v, tgt_v, bkt_v,
             bufs, zidx, zbuf, gsems, wsems, zsem):
  cid = lax.axis_index("c")
  sid = lax.axis_index("s")
  wid = sid * 2 + cid
  b = wid // 2
  half = wid % 2
  mbase = half * HALF
  rowbase = b * M + mbase

  pltpu.sync_copy(dur.at[pl.ds(b * L, L)], dur_v.at[pl.ds(0, L)])
  lanes = lax.iota(jnp.int32, 16)
  carry = jnp.int32(0)
  for i in range(L // 16):
    d = dur_v[pl.ds(i * 16, 16)]
    csum_v[pl.ds(i * 16, 16)] = plsc.cumsum(d) + carry
    carry = carry + jnp.sum(d)
  big = jnp.full((16,), jnp.int32(1 << 30))
  for i in range(L // 16, 2 * L // 16):
    csum_v[pl.ds(i * 16, 16)] = big

  def seg_chunk(i, _):
    m_vec = mbase + i * 16 + lanes
    pos = jnp.zeros((16,), jnp.int32)
    for k in (512, 256, 128, 64, 32, 16, 8, 4, 2, 1):
      cand = pos + k
      vals = plsc.load_gather(csum_v, (cand - 1,))
      pos = jnp.where(vals <= m_vec, cand, pos)
    idx_v[pl.ds(i * 16, 16)] = b * LP + pos
    return 0

  lax.fori_loop(0, NCHUNK, seg_chunk, 0, unroll=4)

  def bucketize(tgt_hbm, bkt_hbm, out_idx):
    pltpu.sync_copy(bkt_hbm, bkt_v)
    pltpu.sync_copy(tgt_hbm.at[pl.ds(b * M + mbase, HALF)], tgt_v)

    def bkt_chunk(i, _):
      t = tgt_v[pl.ds(i * 16, 16)]
      pos = jnp.zeros((16,), jnp.int32)
      for k in (128, 64, 32, 16, 8, 4, 2, 1):
        cand = pos + k
        vals = plsc.load_gather(bkt_v, (cand - 1,))
        pos = jnp.where(vals < t, cand, pos)
      out_idx[pl.ds(i * 16, 16)] = pos
      return 0

    lax.fori_loop(0, NCHUNK, bkt_chunk, 0, unroll=4)

  bucketize(ptgt, pbkt, pidx_v)
  bucketize(etgt, ebkt, eidx_v)
  pltpu.sync_copy(pidx_v, pidx_out.at[pl.ds(rowbase, HALF)])
  pltpu.sync_copy(eidx_v, eidx_out.at[pl.ds(rowbase, HALF)])

  ngood = jnp.clip(carry - mbase, 0, HALF)
  nvc = (ngood + (ROWS - 1)) // ROWS

  for i in range(ROWS // 16):
    zidx[pl.ds(i * 16, 16)] = jnp.full((16,), b * LP + L, jnp.int32)
  pltpu.async_copy(xpad.at[zidx], zbuf, zsem).wait()

  zcopies = [pltpu.make_async_copy(
      zbuf, xm_out.at[pl.ds(rowbase + g * ROWS, ROWS)], zsem)
      for g in range(NGRP)]
  for g in range(NGRP):
    @pl.when(g >= nvc)
    def _(cp=zcopies[g]):
      cp.start()
  for g in range(NGRP):
    @pl.when(g >= nvc)
    def _(cp=zcopies[g]):
      cp.wait()

  copies = []
  for t in range(NGRP):
    s = t % NBUF
    copies.append((
        pltpu.make_async_copy(
            xpad.at[idx_v.at[pl.ds(t * ROWS, ROWS)]], bufs.at[s], gsems[s]),
        pltpu.make_async_copy(
            bufs.at[s], xm_out.at[pl.ds(rowbase + t * ROWS, ROWS)],
            wsems[s])))
  for t in range(NGRP + 1):
    if t < NGRP:
      if t >= NBUF:
        @pl.when(t - NBUF < nvc)
        def _(cp=copies[t - NBUF][1]):
          cp.wait()
      @pl.when(t < nvc)
      def _(cp=copies[t][0]):
        cp.start()
    if t >= 1:
      @pl.when(t - 1 < nvc)
      def _(g=copies[t - 1][0], w=copies[t - 1][1]):
        g.wait()
        w.start()
  for t in range(max(0, NGRP - NBUF), NGRP):
    @pl.when(t < nvc)
    def _(cp=copies[t][1]):
      cp.wait()


def _sc_lr_embed(xpad, dur_flat, ptgt_flat, etgt_flat, pbkt_pad, ebkt_pad):
  mesh = plsc.VectorSubcoreMesh(core_axis_name="c", subcore_axis_name="s")
  f32 = jnp.float32
  run = pl.kernel(
      _sc_body,
      out_type=[jax.ShapeDtypeStruct((B * M, W), f32),
                jax.ShapeDtypeStruct((B * M,), jnp.int32),
                jax.ShapeDtypeStruct((B * M,), jnp.int32)],
      mesh=mesh,
      compiler_params=pltpu.CompilerParams(needs_layout_passes=False),
      scratch_types=[
          pltpu.VMEM((L,), jnp.int32),
          pltpu.VMEM((2 * L,), jnp.int32),
          pltpu.VMEM((HALF,), jnp.int32),
          pltpu.VMEM((HALF,), jnp.int32),
          pltpu.VMEM((HALF,), jnp.int32),
          pltpu.VMEM((HALF,), f32),
          pltpu.VMEM((NB,), f32),
          pltpu.VMEM((NBUF, ROWS, W), f32),
          pltpu.VMEM((ROWS,), jnp.int32),
          pltpu.VMEM((ROWS, W), f32),
          [pltpu.SemaphoreType.DMA] * NBUF,
          [pltpu.SemaphoreType.DMA] * NBUF,
          pltpu.SemaphoreType.DMA,
      ],
  )
  return run(xpad, dur_flat, ptgt_flat, etgt_flat, pbkt_pad, ebkt_pad)


def _conv_ln_relu(x, wk, bias, g, bb):
  x = x.astype(jnp.bfloat16)
  z = jnp.zeros((1, x.shape[1]), x.dtype)
  xdn = jnp.concatenate([z, x[:-1]], axis=0)
  xup = jnp.concatenate([x[1:], z], axis=0)
  y = (jnp.dot(xdn, wk[0], preferred_element_type=jnp.float32)
       + jnp.dot(x, wk[1], preferred_element_type=jnp.float32)
       + jnp.dot(xup, wk[2], preferred_element_type=jnp.float32)
       + bias[0][None, :])
  m = jnp.mean(y, axis=-1, keepdims=True)
  v = jnp.mean((y - m) ** 2, axis=-1, keepdims=True)
  h = (y - m) * lax.rsqrt(v + 1e-5) * g[0][None, :] + bb[0][None, :]
  return jnp.maximum(h, 0.0)


def _pred_tail(h, lw, lb):
  return jnp.maximum(jnp.sum(h * lw[0][None, :], axis=-1) + lb[0, 0], 0.0)


def _dur_body(x_ref, wk1, b1, g1, bb1, wk2, b2, g2, bb2, lw, lb, out_ref):
  h = _conv_ln_relu(x_ref[0], wk1, b1, g1, bb1)
  h = _conv_ln_relu(h, wk2, b2, g2, bb2)
  out_ref[0, 0, :] = _pred_tail(h, lw, lb)


def _emb_rows(idx_row, tab):
  idx_col = jnp.transpose(idx_row, (1, 0))
  oh = (idx_col == lax.broadcasted_iota(jnp.int32, (M, NB), 1))
  return jnp.dot(oh.astype(jnp.bfloat16), tab.astype(jnp.bfloat16),
                 preferred_element_type=jnp.float32)


def _ce_body(mlen_ref, xm_ref, pidx_ref, eidx_ref, ptab_ref, etab_ref,
             pwk1, pb1, pg1, pbb1, pwk2, pb2, pg2, pbb2, plw, plb,
             ewk1, eb1, eg1, ebb1, ewk2, eb2, eg2, ebb2, elw, elb,
             pp_ref, ep_ref, fin_ref):
  frames = lax.broadcasted_iota(jnp.int32, (M, 1), 0)
  xm = jnp.where(frames < mlen_ref[0], xm_ref[0], 0.0)
  h = _conv_ln_relu(xm, pwk1, pb1, pg1, pbb1)
  h = _conv_ln_relu(h, pwk2, pb2, pg2, pbb2)
  pp_ref[0, 0, :] = _pred_tail(h, plw, plb)
  x2 = xm + _emb_rows(pidx_ref[0], ptab_ref[...])
  h = _conv_ln_relu(x2, ewk1, eb1, eg1, ebb1)
  h = _conv_ln_relu(h, ewk2, eb2, eg2, ebb2)
  ep_ref[0, 0, :] = _pred_tail(h, elw, elb)
  fin_ref[0] = x2 + _emb_rows(eidx_ref[0], etab_ref[...])


def _prep(p):
  bf = jnp.bfloat16
  return (jnp.transpose(p['w1'], (2, 1, 0)).astype(bf), p['b1'][None, :],
          p['g1'][None, :], p['bb1'][None, :],
          jnp.transpose(p['w2'], (2, 1, 0)).astype(bf), p['b2'][None, :],
          p['g2'][None, :], p['bb2'][None, :],
          p['lw'], p['lb'][None, :])


def _wspecs():
  full = lambda shape: pl.BlockSpec(shape, lambda b: (0,) * len(shape))
  return [full((K, D, F)), full((1, F)), full((1, F)), full((1, F)),
          full((K, F, F)), full((1, F)), full((1, F)), full((1, F)),
          full((1, F)), full((1, 1))]


def _dur_pred(x, p):
  seq = pl.BlockSpec((1, L, D), lambda b: (b, 0, 0))
  out = pl.pallas_call(
      _dur_body,
      grid=(B,),
      in_specs=[seq] + _wspecs(),
      out_specs=pl.BlockSpec((1, 1, L), lambda b: (b, 0, 0)),
      out_shape=jax.ShapeDtypeStruct((B, 1, L), jnp.float32),
  )(x, *_prep(p))
  return out.reshape(B, L)


def _pitch_energy(max_len, xm, pidx, eidx, ptab, etab, pp, ep):
  seq = pl.BlockSpec((1, M, D), lambda b: (b, 0, 0))
  idxs = pl.BlockSpec((1, 1, M), lambda b: (b, 0, 0))
  tab = pl.BlockSpec((NB, D), lambda b: (0, 0))
  pred = pl.BlockSpec((1, 1, M), lambda b: (b, 0, 0))
  sspec = pl.BlockSpec(memory_space=pltpu.SMEM)
  ppd, epd, fin = pl.pallas_call(
      _ce_body,
      grid=(B,),
      in_specs=[sspec, seq, idxs, idxs, tab, tab] + _wspecs() + _wspecs(),
      out_specs=[pred, pred, seq],
      out_shape=[jax.ShapeDtypeStruct((B, 1, M), jnp.float32),
                 jax.ShapeDtypeStruct((B, 1, M), jnp.float32),
                 jax.ShapeDtypeStruct((B, M, D), jnp.float32)],
  )(jnp.asarray(max_len, jnp.int32).reshape(1), xm, pidx, eidx, ptab, etab,
    *_prep(pp), *_prep(ep))
  return ppd.reshape(B, M), epd.reshape(B, M), fin


def kernel(x, dur_target, pitch_target, energy_target, max_len, mask, params,
           pitch_bucket, energy_bucket):
  f32 = jnp.float32
  xpad = jnp.concatenate([x, jnp.zeros((B, 1, D), f32)], axis=1)
  xpad = xpad.reshape(B * LP, D)
  inf = jnp.full((1,), jnp.inf, f32)
  pbkt_pad = jnp.concatenate([pitch_bucket, inf])
  ebkt_pad = jnp.concatenate([energy_bucket, inf])

  xm, pidx, eidx = _sc_lr_embed(
      xpad, dur_target.reshape(-1), pitch_target.reshape(-1),
      energy_target.reshape(-1), pbkt_pad, ebkt_pad)

  dur_pred = _dur_pred(x, params['dur'])
  pitch_pred, energy_pred, final = _pitch_energy(
      max_len, xm.reshape(B, M, D), pidx.reshape(B, 1, M),
      eidx.reshape(B, 1, M), params['pitch_emb'], params['energy_emb'],
      params['pitch'], params['energy'])
  return (final, dur_pred, pitch_pred, energy_pred)


# speedup vs baseline: 3.3846x; 1.2696x over previous
import functools

import jax
import jax.numpy as jnp
from jax import lax
from jax.experimental import pallas as pl
from jax.experimental.pallas import tpu as pltpu
from jax.experimental.pallas import tpu_sc as plsc

B, L, M, D, F, K, NB = 16, 512, 2048, 256, 256, 3, 256
LP = L + 1
HALF = M // 2
NCHUNK = HALF // 16
ROWS = 64
NGRP = HALF // ROWS
W = D
NBUF = 4


def _sc_body(xpad, dur, ptgt, etgt, pbkt, ebkt,
             xm_out, pidx_out, eidx_out,
             dur_v, csum_v, idx_v, pidx_v, eidx_v, tgt_v, bkt_v,
             bufs, zidx, zbuf, gsems, wsems, zsem):
  cid = lax.axis_index("c")
  sid = lax.axis_index("s")
  wid = sid * 2 + cid
  b = wid // 2
  half = wid % 2
  mbase = half * HALF
  rowbase = b * M + mbase

  pltpu.sync_copy(dur.at[pl.ds(b * L, L)], dur_v.at[pl.ds(0, L)])
  lanes = lax.iota(jnp.int32, 16)
  carry = jnp.int32(0)
  for i in range(L // 16):
    d = dur_v[pl.ds(i * 16, 16)]
    csum_v[pl.ds(i * 16, 16)] = plsc.cumsum(d) + carry
    carry = carry + jnp.sum(d)
  big = jnp.full((16,), jnp.int32(1 << 30))
  for i in range(L // 16, 2 * L // 16):
    csum_v[pl.ds(i * 16, 16)] = big

  def seg_chunk(i, _):
    m_vec = mbase + i * 16 + lanes
    pos = jnp.zeros((16,), jnp.int32)
    for k in (512, 256, 128, 64, 32, 16, 8, 4, 2, 1):
      cand = pos + k
      vals = plsc.load_gather(csum_v, (cand - 1,))
      pos = jnp.where(vals <= m_vec, cand, pos)
    idx_v[pl.ds(i * 16, 16)] = b * LP + pos
    return 0

  lax.fori_loop(0, NCHUNK, seg_chunk, 0, unroll=4)

  def bucketize(tgt_hbm, bkt_hbm, out_idx):
    pltpu.sync_copy(bkt_hbm, bkt_v)
    pltpu.sync_copy(tgt_hbm.at[pl.ds(b * M + mbase, HALF)], tgt_v)

    def bkt_chunk(i, _):
      t = tgt_v[pl.ds(i * 16, 16)]
      pos = jnp.zeros((16,), jnp.int32)
      for k in (128, 64, 32, 16, 8, 4, 2, 1):
        cand = pos + k
        vals = plsc.load_gather(bkt_v, (cand - 1,))
        pos = jnp.where(vals < t, cand, pos)
      out_idx[pl.ds(i * 16, 16)] = pos
      return 0

    lax.fori_loop(0, NCHUNK, bkt_chunk, 0, unroll=4)

  bucketize(ptgt, pbkt, pidx_v)
  bucketize(etgt, ebkt, eidx_v)
  pltpu.sync_copy(pidx_v, pidx_out.at[b, pl.ds(mbase, HALF)])
  pltpu.sync_copy(eidx_v, eidx_out.at[b, pl.ds(mbase, HALF)])

  ngood = jnp.clip(carry - mbase, 0, HALF)
  nvc = (ngood + (ROWS - 1)) // ROWS

  for i in range(ROWS // 16):
    zidx[pl.ds(i * 16, 16)] = jnp.full((16,), b * LP + L, jnp.int32)
  pltpu.async_copy(xpad.at[zidx], zbuf, zsem).wait()

  zcopies = [pltpu.make_async_copy(
      zbuf, xm_out.at[b, pl.ds(mbase + g * ROWS, ROWS)], zsem)
      for g in range(NGRP)]
  for g in range(NGRP):
    @pl.when(g >= nvc)
    def _(cp=zcopies[g]):
      cp.start()
  for g in range(NGRP):
    @pl.when(g >= nvc)
    def _(cp=zcopies[g]):
      cp.wait()

  copies = []
  for t in range(NGRP):
    s = t % NBUF
    copies.append((
        pltpu.make_async_copy(
            xpad.at[idx_v.at[pl.ds(t * ROWS, ROWS)]], bufs.at[s], gsems[s]),
        pltpu.make_async_copy(
            bufs.at[s], xm_out.at[b, pl.ds(mbase + t * ROWS, ROWS)],
            wsems[s])))
  for t in range(NGRP + 1):
    if t < NGRP:
      if t >= NBUF:
        @pl.when(t - NBUF < nvc)
        def _(cp=copies[t - NBUF][1]):
          cp.wait()
      @pl.when(t < nvc)
      def _(cp=copies[t][0]):
        cp.start()
    if t >= 1:
      @pl.when(t - 1 < nvc)
      def _(g=copies[t - 1][0], w=copies[t - 1][1]):
        g.wait()
        w.start()
  for t in range(max(0, NGRP - NBUF), NGRP):
    @pl.when(t < nvc)
    def _(cp=copies[t][1]):
      cp.wait()


def _sc_lr_embed(xpad, dur_flat, ptgt_flat, etgt_flat, pbkt_pad, ebkt_pad):
  mesh = plsc.VectorSubcoreMesh(core_axis_name="c", subcore_axis_name="s")
  f32 = jnp.float32
  run = pl.kernel(
      _sc_body,
      out_type=[jax.ShapeDtypeStruct((B, M, W), f32),
                jax.ShapeDtypeStruct((B, M), jnp.int32),
                jax.ShapeDtypeStruct((B, M), jnp.int32)],
      mesh=mesh,
      compiler_params=pltpu.CompilerParams(needs_layout_passes=False),
      scratch_types=[
          pltpu.VMEM((L,), jnp.int32),
          pltpu.VMEM((2 * L,), jnp.int32),
          pltpu.VMEM((HALF,), jnp.int32),
          pltpu.VMEM((HALF,), jnp.int32),
          pltpu.VMEM((HALF,), jnp.int32),
          pltpu.VMEM((HALF,), f32),
          pltpu.VMEM((NB,), f32),
          pltpu.VMEM((NBUF, ROWS, W), f32),
          pltpu.VMEM((ROWS,), jnp.int32),
          pltpu.VMEM((ROWS, W), f32),
          [pltpu.SemaphoreType.DMA] * NBUF,
          [pltpu.SemaphoreType.DMA] * NBUF,
          pltpu.SemaphoreType.DMA,
      ],
  )
  return run(xpad, dur_flat, ptgt_flat, etgt_flat, pbkt_pad, ebkt_pad)


def _conv_ln_relu(x, wk, bias, g, bb):
  x = x.astype(jnp.bfloat16)
  z = jnp.zeros((1, x.shape[1]), x.dtype)
  xdn = jnp.concatenate([z, x[:-1]], axis=0)
  xup = jnp.concatenate([x[1:], z], axis=0)
  y = (jnp.dot(xdn, wk[0], preferred_element_type=jnp.float32)
       + jnp.dot(x, wk[1], preferred_element_type=jnp.float32)
       + jnp.dot(xup, wk[2], preferred_element_type=jnp.float32)
       + bias[0][None, :])
  m = jnp.mean(y, axis=-1, keepdims=True)
  v = jnp.mean(y * y, axis=-1, keepdims=True) - m * m
  h = (y - m) * lax.rsqrt(v + 1e-5) * g[0][None, :] + bb[0][None, :]
  return jnp.maximum(h, 0.0).astype(jnp.bfloat16)


def _pred_tail(h, lwcol, lb):
  p = jnp.dot(h, lwcol[...], preferred_element_type=jnp.float32)
  return jnp.maximum(jnp.transpose(p, (1, 0))[0] + lb[0, 0], 0.0)


def _dur_body(x_ref, wk1, b1, g1, bb1, wk2, b2, g2, bb2, lw, lb, out_ref):
  h = _conv_ln_relu(x_ref[0], wk1, b1, g1, bb1)
  h = _conv_ln_relu(h, wk2, b2, g2, bb2)
  out_ref[0, 0, :] = _pred_tail(h, lw, lb)


def _emb_rows(idx_row, tab):
  idx_col = jnp.transpose(idx_row, (1, 0))
  oh = (idx_col == lax.broadcasted_iota(jnp.int32, (M, NB), 1))
  return jnp.dot(oh.astype(jnp.bfloat16), tab.astype(jnp.bfloat16),
                 preferred_element_type=jnp.float32)


def _ce_body(mlen_ref, xm_ref, pidx_ref, eidx_ref, ptab_ref, etab_ref,
             pwk1, pb1, pg1, pbb1, pwk2, pb2, pg2, pbb2, plw, plb,
             ewk1, eb1, eg1, ebb1, ewk2, eb2, eg2, ebb2, elw, elb,
             pp_ref, ep_ref, fin_ref):
  frames = lax.broadcasted_iota(jnp.int32, (M, 1), 0)
  xm = jnp.where(frames < mlen_ref[0], xm_ref[0], 0.0)
  h = _conv_ln_relu(xm, pwk1, pb1, pg1, pbb1)
  h = _conv_ln_relu(h, pwk2, pb2, pg2, pbb2)
  pp_ref[0, 0, :] = _pred_tail(h, plw, plb)
  x2 = xm + _emb_rows(pidx_ref[0], ptab_ref[...])
  h = _conv_ln_relu(x2, ewk1, eb1, eg1, ebb1)
  h = _conv_ln_relu(h, ewk2, eb2, eg2, ebb2)
  ep_ref[0, 0, :] = _pred_tail(h, elw, elb)
  fin_ref[0] = x2 + _emb_rows(eidx_ref[0], etab_ref[...])


def _prep(p):
  bf = jnp.bfloat16
  return (jnp.transpose(p['w1'], (2, 1, 0)).astype(bf), p['b1'][None, :],
          p['g1'][None, :], p['bb1'][None, :],
          jnp.transpose(p['w2'], (2, 1, 0)).astype(bf), p['b2'][None, :],
          p['g2'][None, :], p['bb2'][None, :],
          jnp.transpose(p['lw'], (1, 0)).astype(bf), p['lb'][None, :])


def _wspecs():
  full = lambda shape: pl.BlockSpec(shape, lambda b: (0,) * len(shape))
  return [full((K, D, F)), full((1, F)), full((1, F)), full((1, F)),
          full((K, F, F)), full((1, F)), full((1, F)), full((1, F)),
          full((F, 1)), full((1, 1))]


def _dur_pred(x, p):
  seq = pl.BlockSpec((1, L, D), lambda b: (b, 0, 0))
  out = pl.pallas_call(
      _dur_body,
      grid=(B,),
      in_specs=[seq] + _wspecs(),
      out_specs=pl.BlockSpec((1, 1, L), lambda b: (b, 0, 0)),
      out_shape=jax.ShapeDtypeStruct((B, 1, L), jnp.float32),
  )(x, *_prep(p))
  return out.reshape(B, L)


def _pitch_energy(max_len, xm, pidx, eidx, ptab, etab, pp, ep):
  seq = pl.BlockSpec((1, M, D), lambda b: (b, 0, 0))
  idxs = pl.BlockSpec((1, 1, M), lambda b: (b, 0, 0))
  tab = pl.BlockSpec((NB, D), lambda b: (0, 0))
  pred = pl.BlockSpec((1, 1, M), lambda b: (b, 0, 0))
  sspec = pl.BlockSpec(memory_space=pltpu.SMEM)
  ppd, epd, fin = pl.pallas_call(
      _ce_body,
      grid=(B,),
      in_specs=[sspec, seq, idxs, idxs, tab, tab] + _wspecs() + _wspecs(),
      out_specs=[pred, pred, seq],
      out_shape=[jax.ShapeDtypeStruct((B, 1, M), jnp.float32),
                 jax.ShapeDtypeStruct((B, 1, M), jnp.float32),
                 jax.ShapeDtypeStruct((B, M, D), jnp.float32)],
  )(jnp.asarray(max_len, jnp.int32).reshape(1), xm, pidx, eidx, ptab, etab,
    *_prep(pp), *_prep(ep))
  return ppd.reshape(B, M), epd.reshape(B, M), fin


def kernel(x, dur_target, pitch_target, energy_target, max_len, mask, params,
           pitch_bucket, energy_bucket):
  f32 = jnp.float32
  xpad = jnp.concatenate([x, jnp.zeros((B, 1, D), f32)], axis=1)
  xpad = xpad.reshape(B * LP, D)
  inf = jnp.full((1,), jnp.inf, f32)
  pbkt_pad = jnp.concatenate([pitch_bucket, inf])
  ebkt_pad = jnp.concatenate([energy_bucket, inf])

  xm, pidx, eidx = _sc_lr_embed(
      xpad, dur_target.reshape(-1), pitch_target.reshape(-1),
      energy_target.reshape(-1), pbkt_pad, ebkt_pad)

  dur_pred = _dur_pred(x, params['dur'])
  pitch_pred, energy_pred, final = _pitch_energy(
      max_len, xm, pidx.reshape(B, 1, M),
      eidx.reshape(B, 1, M), params['pitch_emb'], params['energy_emb'],
      params['pitch'], params['energy'])
  return (final, dur_pred, pitch_pred, energy_pred)


# speedup vs baseline: 3.4495x; 1.0192x over previous
import functools

import jax
import jax.numpy as jnp
from jax import lax
from jax.experimental import pallas as pl
from jax.experimental.pallas import tpu as pltpu
from jax.experimental.pallas import tpu_sc as plsc

B, L, M, D, F, K, NB = 16, 512, 2048, 256, 256, 3, 256
LP = L + 1
HALF = M // 2
NCHUNK = HALF // 16
ROWS = 64
NGRP = HALF // ROWS
W = D
NBUF = 4


def _sc_body(xpad, dur, ptgt, etgt, pbkt, ebkt,
             xm_out, pidx_out, eidx_out,
             dur_v, csum_v, idx_v, pidx_v, eidx_v, tgt_v, bkt_v,
             bufs, zidx, zbuf, gsems, wsems, zsem):
  cid = lax.axis_index("c")
  sid = lax.axis_index("s")
  wid = sid * 2 + cid
  b = wid // 2
  half = wid % 2
  mbase = half * HALF
  rowbase = b * M + mbase

  pltpu.sync_copy(dur.at[pl.ds(b * L, L)], dur_v.at[pl.ds(0, L)])
  lanes = lax.iota(jnp.int32, 16)
  carry = jnp.int32(0)
  for i in range(L // 16):
    d = dur_v[pl.ds(i * 16, 16)]
    csum_v[pl.ds(i * 16, 16)] = plsc.cumsum(d) + carry
    carry = carry + jnp.sum(d)
  big = jnp.full((16,), jnp.int32(1 << 30))
  for i in range(L // 16, 2 * L // 16):
    csum_v[pl.ds(i * 16, 16)] = big
  ngood = jnp.clip(carry - mbase, 0, HALF)
  nvc = (ngood + (ROWS - 1)) // ROWS

  def seg_chunk(i, _):
    m_vec = mbase + i * 16 + lanes
    pos = jnp.zeros((16,), jnp.int32)
    for k in (512, 256, 128, 64, 32, 16, 8, 4, 2, 1):
      cand = pos + k
      vals = plsc.load_gather(csum_v, (cand - 1,))
      pos = jnp.where(vals <= m_vec, cand, pos)
    idx_v[pl.ds(i * 16, 16)] = b * LP + pos
    return 0

  lax.fori_loop(0, nvc * (ROWS // 16), seg_chunk, 0)

  def bucketize(tgt_hbm, bkt_hbm, out_idx):
    pltpu.sync_copy(bkt_hbm, bkt_v)
    pltpu.sync_copy(tgt_hbm.at[pl.ds(b * M + mbase, HALF)], tgt_v)

    def bkt_chunk(i, _):
      t = tgt_v[pl.ds(i * 16, 16)]
      pos = jnp.zeros((16,), jnp.int32)
      for k in (128, 64, 32, 16, 8, 4, 2, 1):
        cand = pos + k
        vals = plsc.load_gather(bkt_v, (cand - 1,))
        pos = jnp.where(vals < t, cand, pos)
      out_idx[pl.ds(i * 16, 16)] = pos
      return 0

    lax.fori_loop(0, NCHUNK, bkt_chunk, 0, unroll=4)

  bucketize(ptgt, pbkt, pidx_v)
  bucketize(etgt, ebkt, eidx_v)
  pltpu.sync_copy(pidx_v, pidx_out.at[b, pl.ds(mbase, HALF)])
  pltpu.sync_copy(eidx_v, eidx_out.at[b, pl.ds(mbase, HALF)])

  for i in range(ROWS // 16):
    zidx[pl.ds(i * 16, 16)] = jnp.full((16,), b * LP + L, jnp.int32)
  pltpu.async_copy(xpad.at[zidx], zbuf, zsem).wait()

  zcopies = [pltpu.make_async_copy(
      zbuf, xm_out.at[b, pl.ds(mbase + g * ROWS, ROWS)], zsem)
      for g in range(NGRP)]
  for g in range(NGRP):
    @pl.when(g >= nvc)
    def _(cp=zcopies[g]):
      cp.start()
  for g in range(NGRP):
    @pl.when(g >= nvc)
    def _(cp=zcopies[g]):
      cp.wait()

  copies = []
  for t in range(NGRP):
    s = t % NBUF
    copies.append((
        pltpu.make_async_copy(
            xpad.at[idx_v.at[pl.ds(t * ROWS, ROWS)]], bufs.at[s], gsems[s]),
        pltpu.make_async_copy(
            bufs.at[s], xm_out.at[b, pl.ds(mbase + t * ROWS, ROWS)],
            wsems[s])))
  for t in range(NGRP + 1):
    if t < NGRP:
      if t >= NBUF:
        @pl.when(t - NBUF < nvc)
        def _(cp=copies[t - NBUF][1]):
          cp.wait()
      @pl.when(t < nvc)
      def _(cp=copies[t][0]):
        cp.start()
    if t >= 1:
      @pl.when(t - 1 < nvc)
      def _(g=copies[t - 1][0], w=copies[t - 1][1]):
        g.wait()
        w.start()
  for t in range(max(0, NGRP - NBUF), NGRP):
    @pl.when(t < nvc)
    def _(cp=copies[t][1]):
      cp.wait()


def _sc_lr_embed(xpad, dur_flat, ptgt_flat, etgt_flat, pbkt_pad, ebkt_pad):
  mesh = plsc.VectorSubcoreMesh(core_axis_name="c", subcore_axis_name="s")
  f32 = jnp.float32
  run = pl.kernel(
      _sc_body,
      out_type=[jax.ShapeDtypeStruct((B, M, W), f32),
                jax.ShapeDtypeStruct((B, M), jnp.int32),
                jax.ShapeDtypeStruct((B, M), jnp.int32)],
      mesh=mesh,
      compiler_params=pltpu.CompilerParams(needs_layout_passes=False),
      scratch_types=[
          pltpu.VMEM((L,), jnp.int32),
          pltpu.VMEM((2 * L,), jnp.int32),
          pltpu.VMEM((HALF,), jnp.int32),
          pltpu.VMEM((HALF,), jnp.int32),
          pltpu.VMEM((HALF,), jnp.int32),
          pltpu.VMEM((HALF,), f32),
          pltpu.VMEM((NB,), f32),
          pltpu.VMEM((NBUF, ROWS, W), f32),
          pltpu.VMEM((ROWS,), jnp.int32),
          pltpu.VMEM((ROWS, W), f32),
          [pltpu.SemaphoreType.DMA] * NBUF,
          [pltpu.SemaphoreType.DMA] * NBUF,
          pltpu.SemaphoreType.DMA,
      ],
  )
  return run(xpad, dur_flat, ptgt_flat, etgt_flat, pbkt_pad, ebkt_pad)


def _conv_ln_relu(x, wk, bias, g, bb):
  x = x.astype(jnp.bfloat16)
  z = jnp.zeros((1, x.shape[1]), x.dtype)
  xdn = jnp.concatenate([z, x[:-1]], axis=0)
  xup = jnp.concatenate([x[1:], z], axis=0)
  y = (jnp.dot(xdn, wk[0], preferred_element_type=jnp.float32)
       + jnp.dot(x, wk[1], preferred_element_type=jnp.float32)
       + jnp.dot(xup, wk[2], preferred_element_type=jnp.float32)
       + bias[0][None, :])
  m = jnp.mean(y, axis=-1, keepdims=True)
  v = jnp.mean(y * y, axis=-1, keepdims=True) - m * m
  h = (y - m) * lax.rsqrt(v + 1e-5) * g[0][None, :] + bb[0][None, :]
  return jnp.maximum(h, 0.0).astype(jnp.bfloat16)


def _pred_tail(h, lwcol, lb):
  p = jnp.dot(h, lwcol[...], preferred_element_type=jnp.float32)
  return jnp.maximum(jnp.transpose(p, (1, 0))[0] + lb[0, 0], 0.0)


def _dur_body(x_ref, wk1, b1, g1, bb1, wk2, b2, g2, bb2, lw, lb, out_ref):
  h = _conv_ln_relu(x_ref[0], wk1, b1, g1, bb1)
  h = _conv_ln_relu(h, wk2, b2, g2, bb2)
  out_ref[0, 0, :] = _pred_tail(h, lw, lb)


def _emb_rows(idx_row, tab):
  idx_col = jnp.transpose(idx_row, (1, 0))
  oh = (idx_col == lax.broadcasted_iota(jnp.int32, (M, NB), 1))
  return jnp.dot(oh.astype(jnp.bfloat16), tab.astype(jnp.bfloat16),
                 preferred_element_type=jnp.float32)


def _ce_body(mlen_ref, xm_ref, pidx_ref, eidx_ref, ptab_ref, etab_ref,
             pwk1, pb1, pg1, pbb1, pwk2, pb2, pg2, pbb2, plw, plb,
             ewk1, eb1, eg1, ebb1, ewk2, eb2, eg2, ebb2, elw, elb,
             pp_ref, ep_ref, fin_ref):
  frames = lax.broadcasted_iota(jnp.int32, (M, 1), 0)
  xm = jnp.where(frames < mlen_ref[0], xm_ref[0], 0.0)
  h = _conv_ln_relu(xm, pwk1, pb1, pg1, pbb1)
  h = _conv_ln_relu(h, pwk2, pb2, pg2, pbb2)
  pp_ref[0, 0, :] = _pred_tail(h, plw, plb)
  x2 = xm + _emb_rows(pidx_ref[0], ptab_ref[...])
  h = _conv_ln_relu(x2, ewk1, eb1, eg1, ebb1)
  h = _conv_ln_relu(h, ewk2, eb2, eg2, ebb2)
  ep_ref[0, 0, :] = _pred_tail(h, elw, elb)
  fin_ref[0] = x2 + _emb_rows(eidx_ref[0], etab_ref[...])


def _prep(p):
  bf = jnp.bfloat16
  wc = lambda w: jnp.transpose(w, (2, 1, 0)).astype(bf)
  return (wc(p['w1']), p['b1'][None, :],
          p['g1'][None, :], p['bb1'][None, :],
          wc(p['w2']), p['b2'][None, :],
          p['g2'][None, :], p['bb2'][None, :],
          jnp.transpose(p['lw'], (1, 0)).astype(bf), p['lb'][None, :])


def _wspecs():
  full = lambda shape: pl.BlockSpec(shape, lambda b: (0,) * len(shape))
  return [full((K, D, F)), full((1, F)), full((1, F)), full((1, F)),
          full((K, F, F)), full((1, F)), full((1, F)), full((1, F)),
          full((F, 1)), full((1, 1))]


def _dur_pred(x, p):
  seq = pl.BlockSpec((1, L, D), lambda b: (b, 0, 0))
  out = pl.pallas_call(
      _dur_body,
      grid=(B,),
      in_specs=[seq] + _wspecs(),
      out_specs=pl.BlockSpec((1, 1, L), lambda b: (b, 0, 0)),
      out_shape=jax.ShapeDtypeStruct((B, 1, L), jnp.float32),
  )(x, *_prep(p))
  return out.reshape(B, L)


def _pitch_energy(max_len, xm, pidx, eidx, ptab, etab, pp, ep):
  seq = pl.BlockSpec((1, M, D), lambda b: (b, 0, 0))
  idxs = pl.BlockSpec((1, 1, M), lambda b: (b, 0, 0))
  tab = pl.BlockSpec((NB, D), lambda b: (0, 0))
  pred = pl.BlockSpec((1, 1, M), lambda b: (b, 0, 0))
  sspec = pl.BlockSpec(memory_space=pltpu.SMEM)
  ppd, epd, fin = pl.pallas_call(
      _ce_body,
      grid=(B,),
      in_specs=[sspec, seq, idxs, idxs, tab, tab] + _wspecs() + _wspecs(),
      out_specs=[pred, pred, seq],
      out_shape=[jax.ShapeDtypeStruct((B, 1, M), jnp.float32),
                 jax.ShapeDtypeStruct((B, 1, M), jnp.float32),
                 jax.ShapeDtypeStruct((B, M, D), jnp.float32)],
  )(jnp.asarray(max_len, jnp.int32).reshape(1), xm, pidx, eidx, ptab, etab,
    *_prep(pp), *_prep(ep))
  return ppd.reshape(B, M), epd.reshape(B, M), fin


def kernel(x, dur_target, pitch_target, energy_target, max_len, mask, params,
           pitch_bucket, energy_bucket):
  f32 = jnp.float32
  xpad = jnp.concatenate([x, jnp.zeros((B, 1, D), f32)], axis=1)
  xpad = xpad.reshape(B * LP, D)
  inf = jnp.full((1,), jnp.inf, f32)
  pbkt_pad = jnp.concatenate([pitch_bucket, inf])
  ebkt_pad = jnp.concatenate([energy_bucket, inf])

  xm, pidx, eidx = _sc_lr_embed(
      xpad, dur_target.reshape(-1), pitch_target.reshape(-1),
      energy_target.reshape(-1), pbkt_pad, ebkt_pad)

  dur_pred = _dur_pred(x, params['dur'])
  pitch_pred, energy_pred, final = _pitch_energy(
      max_len, xm, pidx.reshape(B, 1, M),
      eidx.reshape(B, 1, M), params['pitch_emb'], params['energy_emb'],
      params['pitch'], params['energy'])
  return (final, dur_pred, pitch_pred, energy_pred)
